# Initial kernel scaffold; baseline (speedup 1.0000x reference)
import jax
import jax.numpy as jnp
from jax.experimental import pallas as pl


def kernel(user, item_i, item_j, user_emb, item_emb, adj_row, adj_col, adj_val, W_gc_0, b_gc_0, W_bi_0, b_bi_0, W_gc_1, b_gc_1, W_bi_1, b_bi_1, W_gc_2, b_gc_2, W_bi_2, b_bi_2):
    raise NotImplementedError("write your pallas kernel here")


# speedup vs baseline: 4.1264x; 4.1264x over previous
import functools

import jax
import jax.numpy as jnp
from jax import lax
from jax.experimental import pallas as pl
from jax.experimental.pallas import tpu as pltpu
from jax.experimental.pallas import tpu_sc as plsc

NUSER = 30000
NITEM = 70000
NN = NUSER + NITEM
NP = 100096
EE = 1600000
DD = 32
HALF = 16
BB = 4096

NC = 2
NS = 16
EPT = EE // NS
CHUNK = 800
NCHUNK = EPT // CHUNK
GG = 80
NG = CHUNK // GG
RPT = NP // NS
RBLK = 4000
GPW = BB // (NC * NS)


def _spmv_body(tbl, colflat, rowflat, val, out, colv, valv, rowfv, rowsv, acc, sem):
    c = lax.axis_index("c")
    s = lax.axis_index("s")

    def zero_rows(i, carry):
        rowsv[i, :] = jnp.zeros((HALF,), jnp.float32)
        return carry

    lax.fori_loop(0, CHUNK, zero_rows, 0)

    zbase = s * RPT
    for q in range(RPT // CHUNK):
        pltpu.sync_copy(rowsv, acc.at[pl.ds(zbase + q * CHUNK, CHUNK)])
    rem = RPT - (RPT // CHUNK) * CHUNK
    if rem:
        pltpu.sync_copy(rowsv.at[pl.ds(0, rem)],
                        acc.at[pl.ds(zbase + (RPT // CHUNK) * CHUNK, rem)])
    plsc.subcore_barrier()

    def chunk_body(k, carry):
        base = s * EPT + k * CHUNK
        pltpu.sync_copy(colflat.at[pl.ds(c * EE + base, CHUNK)], colv)
        pltpu.sync_copy(val.at[pl.ds(base, CHUNK)], valv)
        pltpu.sync_copy(rowflat.at[pl.ds(base, CHUNK)], rowfv)

        def grp(j, gcarry):
            off = j * GG
            pltpu.async_copy(tbl.at[colv.at[pl.ds(off, GG)]],
                             rowsv.at[pl.ds(off, GG)], sem).wait()
            for t in range(GG // HALF):
                vv = valv[pl.ds(off + t * HALF, HALF)]
                for u in range(HALF):
                    i = off + t * HALF + u
                    rowsv[i, :] = rowsv[i, :] * vv[u]
            pltpu.sync_copy(rowsv.at[pl.ds(off, GG)],
                            acc.at[rowfv.at[pl.ds(off, GG)]], add=True)
            return gcarry

        lax.fori_loop(0, NG, grp, 0)
        return carry

    lax.fori_loop(0, NCHUNK, chunk_body, 0)
    plsc.subcore_barrier()
    obase = c * NP + s * RPT
    for q in range(RPT // CHUNK):
        pltpu.sync_copy(acc.at[pl.ds(zbase + q * CHUNK, CHUNK)],
                        out.at[pl.ds(obase + q * CHUNK, CHUNK)])
    if rem:
        pltpu.sync_copy(acc.at[pl.ds(zbase + (RPT // CHUNK) * CHUNK, rem)],
                        out.at[pl.ds(obase + (RPT // CHUNK) * CHUNK, rem)])


@functools.cache
def _make_spmv():
    return functools.partial(
        pl.kernel,
        out_type=jax.ShapeDtypeStruct((2 * NP, HALF), jnp.float32),
        mesh=plsc.VectorSubcoreMesh(core_axis_name="c", subcore_axis_name="s",
                                    num_cores=NC, num_subcores=NS),
        scratch_types=[
            pltpu.VMEM((CHUNK,), jnp.int32),
            pltpu.VMEM((CHUNK,), jnp.float32),
            pltpu.VMEM((CHUNK,), jnp.int32),
            pltpu.VMEM((CHUNK, HALF), jnp.float32),
            pltpu.VMEM_SHARED((NP, HALF), jnp.float32),
            pltpu.SemaphoreType.DMA,
        ],
        compiler_params=pltpu.CompilerParams(use_tc_tiling_on_sc=False),
    )(_spmv_body)


def _dense_body(sL, sR, eL, eR, wg_ref, bg_ref, wb_ref, bb_ref, ego_out, norm_out):
    sl = sL[...]
    sr = sR[...]
    el = eL[...]
    er = eR[...]
    wg = wg_ref[...]
    wb = wb_ref[...]
    x = (jnp.dot(sl, wg[:HALF, :], preferred_element_type=jnp.float32)
         + jnp.dot(sr, wg[HALF:, :], preferred_element_type=jnp.float32)
         + jnp.dot(el * sl, wb[:HALF, :], preferred_element_type=jnp.float32)
         + jnp.dot(er * sr, wb[HALF:, :], preferred_element_type=jnp.float32)
         + bg_ref[...] + bb_ref[...])
    act = jnp.where(x >= 0, x, 0.2 * x)
    ss = jnp.sum(act * act, axis=1, keepdims=True)
    denom = jnp.maximum(jnp.sqrt(ss), 1e-12)
    norm_out[...] = act / denom
    ego_out[0, :, :] = act[:, :HALF]
    ego_out[1, :, :] = act[:, HALF:]


_dense = pl.pallas_call(
    _dense_body,
    grid=(NN // RBLK,),
    in_specs=[
        pl.BlockSpec((RBLK, HALF), lambda i: (i, 0)),
        pl.BlockSpec((RBLK, HALF), lambda i: (i, 0)),
        pl.BlockSpec((RBLK, HALF), lambda i: (i, 0)),
        pl.BlockSpec((RBLK, HALF), lambda i: (i, 0)),
        pl.BlockSpec((DD, DD), lambda i: (0, 0)),
        pl.BlockSpec((1, DD), lambda i: (0, 0)),
        pl.BlockSpec((DD, DD), lambda i: (0, 0)),
        pl.BlockSpec((1, DD), lambda i: (0, 0)),
    ],
    out_specs=[
        pl.BlockSpec((2, RBLK, HALF), lambda i: (0, i, 0)),
        pl.BlockSpec((RBLK, DD), lambda i: (i, 0)),
    ],
    out_shape=[
        jax.ShapeDtypeStruct((2, NN, HALF), jnp.float32),
        jax.ShapeDtypeStruct((NN, DD), jnp.float32),
    ],
)


def _gather_body(t0, t1, t2, t3, idxflat, out, idxv, buf, sem):
    c = lax.axis_index("c")
    s = lax.axis_index("s")
    base = (s * NC + c) * GPW
    tabs = [t0, t1, t2, t3]
    for kset in range(3):
        pltpu.sync_copy(idxflat.at[pl.ds(kset * BB + base, GPW)], idxv)
        for t in range(4):
            pltpu.async_copy(tabs[t].at[idxv], buf, sem).wait()
            pltpu.sync_copy(buf, out.at[kset, t, pl.ds(base, GPW)])


@functools.cache
def _make_batch_gather():
    return functools.partial(
        pl.kernel,
        out_type=jax.ShapeDtypeStruct((3, 4, BB, DD), jnp.float32),
        mesh=plsc.VectorSubcoreMesh(core_axis_name="c", subcore_axis_name="s",
                                    num_cores=NC, num_subcores=NS),
        scratch_types=[
            pltpu.VMEM((GPW,), jnp.int32),
            pltpu.VMEM((GPW, DD), jnp.float32),
            pltpu.SemaphoreType.DMA,
        ],
        compiler_params=pltpu.CompilerParams(use_tc_tiling_on_sc=False),
    )(_gather_body)


def kernel(user, item_i, item_j, user_emb, item_emb, adj_row, adj_col, adj_val,
           W_gc_0, b_gc_0, W_bi_0, b_bi_0,
           W_gc_1, b_gc_1, W_bi_1, b_bi_1,
           W_gc_2, b_gc_2, W_bi_2, b_bi_2):
    Ws = [(W_gc_0, b_gc_0, W_bi_0, b_bi_0),
          (W_gc_1, b_gc_1, W_bi_1, b_bi_1),
          (W_gc_2, b_gc_2, W_bi_2, b_bi_2)]
    ego0 = jnp.concatenate([user_emb, item_emb], axis=0)
    ego_flat = ego0.reshape(NN, 2, HALF).transpose(1, 0, 2).reshape(2 * NN, HALF)
    colflat = jnp.concatenate([adj_col, adj_col + NN])
    idxflat = jnp.concatenate([user.astype(jnp.int32),
                               item_i + NUSER,
                               item_j + NUSER])

    spmv = _make_spmv()
    norms = []
    for (wg, bg, wb, bb) in Ws:
        side_pad = spmv(ego_flat, colflat, adj_row, adj_val)
        ego_split, norm = _dense(side_pad[:NN], side_pad[NP:NP + NN],
                                 ego_flat[:NN], ego_flat[NN:],
                                 wg, bg, wb, bb)
        ego_flat = ego_split.reshape(2 * NN, HALF)
        norms.append(norm)

    out = _make_batch_gather()(ego0, norms[0], norms[1], norms[2], idxflat)
    u_out = out[0].transpose(1, 0, 2).reshape(BB, 4 * DD)
    pos_out = out[1].transpose(1, 0, 2).reshape(BB, 4 * DD)
    neg_out = out[2].transpose(1, 0, 2).reshape(BB, 4 * DD)
    return (u_out, pos_out, neg_out)


# speedup vs baseline: 10.6050x; 2.5701x over previous
import functools

import jax
import jax.numpy as jnp
from jax import lax
from jax.experimental import pallas as pl
from jax.experimental.pallas import tpu as pltpu
from jax.experimental.pallas import tpu_sc as plsc

NUSER = 30000
NITEM = 70000
NN = NUSER + NITEM
NP = 100096
EE = 1600000
DD = 32
HALF = 16
BB = 4096

NC = 2
NS = 16
EPT = EE // NS
CHUNK = 400
NCHUNK = EPT // CHUNK
GG = 80
NG = CHUNK // GG
RPT = NP // NS
RBLK = 4000
GPW = BB // (NC * NS)


def _spmv_body(tbl, col2, row2, val2, out,
               col_a, col_b, val_a, val_b, rowf_a, rowf_b, rows_a, rows_b,
               acc, semg_a, semg_b, semi_a, semi_b, sems_a, sems_b):
    c = lax.axis_index("c")
    s = lax.axis_index("s")
    cols = [col_a, col_b]
    vals = [val_a, val_b]
    rowfs = [rowf_a, rowf_b]
    rowss = [rows_a, rows_b]
    semgs = [semg_a, semg_b]
    semis = [semi_a, semi_b]
    semss = [sems_a, sems_b]

    def zero_rows(i, carry):
        rows_a[i, :] = jnp.zeros((HALF,), jnp.float32)
        return carry

    lax.fori_loop(0, CHUNK, zero_rows, 0)

    zbase = s * RPT
    for q in range(RPT // CHUNK):
        pltpu.sync_copy(rows_a, acc.at[pl.ds(zbase + q * CHUNK, CHUNK)])
    rem = RPT - (RPT // CHUNK) * CHUNK
    if rem:
        pltpu.sync_copy(rows_a.at[pl.ds(0, rem)],
                        acc.at[pl.ds(zbase + (RPT // CHUNK) * CHUNK, rem)])
    plsc.subcore_barrier()

    def fire_idx_and_gather(k, t):
        rb = s * (EPT // GG) + k * NG
        pltpu.async_copy(col2.at[pl.ds(c * (EE // GG) + rb, NG)], cols[t],
                         semis[t])
        pltpu.async_copy(val2.at[pl.ds(rb, NG)], vals[t], semis[t])
        pltpu.async_copy(row2.at[pl.ds(rb, NG)], rowfs[t], semis[t])
        pltpu.make_async_copy(col2.at[pl.ds(0, NG)], cols[t], semis[t]).wait()
        pltpu.make_async_copy(val2.at[pl.ds(0, NG)], vals[t], semis[t]).wait()
        pltpu.make_async_copy(row2.at[pl.ds(0, NG)], rowfs[t], semis[t]).wait()
        for j in range(NG):
            pltpu.async_copy(tbl.at[cols[t].at[j]],
                             rowss[t].at[pl.ds(j * GG, GG)], semgs[t])

    fire_idx_and_gather(0, 0)

    def outer(i, carry):
        for b in range(2):
            o = 1 - b
            k = 2 * i + b
            @pl.when(k >= 1)
            def _drain_scatter():
                pltpu.make_async_copy(tbl.at[pl.ds(0, CHUNK)], rowss[o],
                                      semss[o]).wait()

            @pl.when(k + 1 < NCHUNK)
            def _prefetch():
                fire_idx_and_gather(k + 1, o)

            pltpu.make_async_copy(tbl.at[pl.ds(0, CHUNK)], rowss[b],
                                  semgs[b]).wait()
            for j in range(NG):
                for t in range(GG // HALF):
                    vv = vals[b][j, pl.ds(t * HALF, HALF)]
                    for u in range(HALF):
                        r = j * GG + t * HALF + u
                        rowss[b][r, :] = rowss[b][r, :] * vv[u]
            for j in range(NG):
                pltpu.async_copy(rowss[b].at[pl.ds(j * GG, GG)],
                                 acc.at[rowfs[b].at[j]], semss[b], add=True)
        return carry

    lax.fori_loop(0, NCHUNK // 2, outer, 0)
    pltpu.make_async_copy(tbl.at[pl.ds(0, CHUNK)], rowss[(NCHUNK - 1) % 2],
                          semss[(NCHUNK - 1) % 2]).wait()
    plsc.subcore_barrier()
    obase = c * NP + s * RPT
    for q in range(RPT // CHUNK):
        pltpu.sync_copy(acc.at[pl.ds(zbase + q * CHUNK, CHUNK)],
                        out.at[pl.ds(obase + q * CHUNK, CHUNK)])
    if rem:
        pltpu.sync_copy(acc.at[pl.ds(zbase + (RPT // CHUNK) * CHUNK, rem)],
                        out.at[pl.ds(obase + (RPT // CHUNK) * CHUNK, rem)])


@functools.cache
def _make_spmv():
    return functools.partial(
        pl.kernel,
        out_type=jax.ShapeDtypeStruct((2 * NP, HALF), jnp.float32),
        mesh=plsc.VectorSubcoreMesh(core_axis_name="c", subcore_axis_name="s",
                                    num_cores=NC, num_subcores=NS),
        scratch_types=[
            pltpu.VMEM((NG, GG), jnp.int32),
            pltpu.VMEM((NG, GG), jnp.int32),
            pltpu.VMEM((NG, GG), jnp.float32),
            pltpu.VMEM((NG, GG), jnp.float32),
            pltpu.VMEM((NG, GG), jnp.int32),
            pltpu.VMEM((NG, GG), jnp.int32),
            pltpu.VMEM((CHUNK, HALF), jnp.float32),
            pltpu.VMEM((CHUNK, HALF), jnp.float32),
            pltpu.VMEM_SHARED((NP, HALF), jnp.float32),
            pltpu.SemaphoreType.DMA,
            pltpu.SemaphoreType.DMA,
            pltpu.SemaphoreType.DMA,
            pltpu.SemaphoreType.DMA,
            pltpu.SemaphoreType.DMA,
            pltpu.SemaphoreType.DMA,
        ],
        compiler_params=pltpu.CompilerParams(use_tc_tiling_on_sc=False),
    )(_spmv_body)


def _dense_body(sL, sR, eL, eR, wg_ref, bg_ref, wb_ref, bb_ref, ego_out, norm_out):
    sl = sL[...]
    sr = sR[...]
    el = eL[...]
    er = eR[...]
    wg = wg_ref[...]
    wb = wb_ref[...]
    x = (jnp.dot(sl, wg[:HALF, :], preferred_element_type=jnp.float32)
         + jnp.dot(sr, wg[HALF:, :], preferred_element_type=jnp.float32)
         + jnp.dot(el * sl, wb[:HALF, :], preferred_element_type=jnp.float32)
         + jnp.dot(er * sr, wb[HALF:, :], preferred_element_type=jnp.float32)
         + bg_ref[...] + bb_ref[...])
    act = jnp.where(x >= 0, x, 0.2 * x)
    ss = jnp.sum(act * act, axis=1, keepdims=True)
    denom = jnp.maximum(jnp.sqrt(ss), 1e-12)
    norm_out[...] = act / denom
    ego_out[0, :, :] = act[:, :HALF]
    ego_out[1, :, :] = act[:, HALF:]


_dense = pl.pallas_call(
    _dense_body,
    grid=(NN // RBLK,),
    in_specs=[
        pl.BlockSpec((RBLK, HALF), lambda i: (i, 0)),
        pl.BlockSpec((RBLK, HALF), lambda i: (i, 0)),
        pl.BlockSpec((RBLK, HALF), lambda i: (i, 0)),
        pl.BlockSpec((RBLK, HALF), lambda i: (i, 0)),
        pl.BlockSpec((DD, DD), lambda i: (0, 0)),
        pl.BlockSpec((1, DD), lambda i: (0, 0)),
        pl.BlockSpec((DD, DD), lambda i: (0, 0)),
        pl.BlockSpec((1, DD), lambda i: (0, 0)),
    ],
    out_specs=[
        pl.BlockSpec((2, RBLK, HALF), lambda i: (0, i, 0)),
        pl.BlockSpec((RBLK, DD), lambda i: (i, 0)),
    ],
    out_shape=[
        jax.ShapeDtypeStruct((2, NN, HALF), jnp.float32),
        jax.ShapeDtypeStruct((NN, DD), jnp.float32),
    ],
)


def _gather_body(t0, t1, t2, t3, idxflat, out, idxv, buf, sem):
    c = lax.axis_index("c")
    s = lax.axis_index("s")
    base = (s * NC + c) * GPW
    tabs = [t0, t1, t2, t3]
    for kset in range(3):
        pltpu.sync_copy(idxflat.at[pl.ds(kset * BB + base, GPW)], idxv)
        for t in range(4):
            pltpu.async_copy(tabs[t].at[idxv], buf, sem).wait()
            pltpu.sync_copy(buf, out.at[kset, t, pl.ds(base, GPW)])


@functools.cache
def _make_batch_gather():
    return functools.partial(
        pl.kernel,
        out_type=jax.ShapeDtypeStruct((3, 4, BB, DD), jnp.float32),
        mesh=plsc.VectorSubcoreMesh(core_axis_name="c", subcore_axis_name="s",
                                    num_cores=NC, num_subcores=NS),
        scratch_types=[
            pltpu.VMEM((GPW,), jnp.int32),
            pltpu.VMEM((GPW, DD), jnp.float32),
            pltpu.SemaphoreType.DMA,
        ],
        compiler_params=pltpu.CompilerParams(use_tc_tiling_on_sc=False),
    )(_gather_body)


def kernel(user, item_i, item_j, user_emb, item_emb, adj_row, adj_col, adj_val,
           W_gc_0, b_gc_0, W_bi_0, b_bi_0,
           W_gc_1, b_gc_1, W_bi_1, b_bi_1,
           W_gc_2, b_gc_2, W_bi_2, b_bi_2):
    Ws = [(W_gc_0, b_gc_0, W_bi_0, b_bi_0),
          (W_gc_1, b_gc_1, W_bi_1, b_bi_1),
          (W_gc_2, b_gc_2, W_bi_2, b_bi_2)]
    ego0 = jnp.concatenate([user_emb, item_emb], axis=0)
    ego_flat = ego0.reshape(NN, 2, HALF).transpose(1, 0, 2).reshape(2 * NN, HALF)
    col2 = jnp.concatenate([adj_col, adj_col + NN]).reshape(2 * EE // GG, GG)
    row2 = adj_row.reshape(EE // GG, GG)
    val2 = adj_val.reshape(EE // GG, GG)
    idxflat = jnp.concatenate([user.astype(jnp.int32),
                               item_i + NUSER,
                               item_j + NUSER])

    spmv = _make_spmv()
    norms = []
    for (wg, bg, wb, bb) in Ws:
        side_pad = spmv(ego_flat, col2, row2, val2)
        ego_split, norm = _dense(side_pad[:NN], side_pad[NP:NP + NN],
                                 ego_flat[:NN], ego_flat[NN:],
                                 wg, bg, wb, bb)
        ego_flat = ego_split.reshape(2 * NN, HALF)
        norms.append(norm)

    out = _make_batch_gather()(ego0, norms[0], norms[1], norms[2], idxflat)
    u_out = out[0].transpose(1, 0, 2).reshape(BB, 4 * DD)
    pos_out = out[1].transpose(1, 0, 2).reshape(BB, 4 * DD)
    neg_out = out[2].transpose(1, 0, 2).reshape(BB, 4 * DD)
    return (u_out, pos_out, neg_out)


# speedup vs baseline: 17.4092x; 1.6416x over previous
import functools

import jax
import jax.numpy as jnp
from jax import lax
from jax.experimental import pallas as pl
from jax.experimental.pallas import tpu as pltpu
from jax.experimental.pallas import tpu_sc as plsc

NUSER = 30000
NITEM = 70000
NN = NUSER + NITEM
NP = 100096
EE = 1600000
DD = 32
HALF = 16
BB = 4096

NC = 2
NS = 16
EPT = EE // NS
CHUNK = 400
NCHUNK = EPT // CHUNK
GG = 80
NG = CHUNK // GG
RPT = NP // NS
RR = NP // 8
RB = 3128
GPW = BB // (NC * NS)


def _spmv_body(tbl, col2, row2, val2, out,
               col_a, col_b, val_a, val_b, rowf_a, rowf_b, rows_a, rows_b,
               acc, semg_a, semg_b, semi_a, semi_b, sems_a, sems_b):
    c = lax.axis_index("c")
    s = lax.axis_index("s")
    cols = [col_a, col_b]
    vals = [val_a, val_b]
    rowfs = [rowf_a, rowf_b]
    rowss = [rows_a, rows_b]
    semgs = [semg_a, semg_b]
    semis = [semi_a, semi_b]
    semss = [sems_a, sems_b]

    def zero_rows(i, carry):
        rows_a[i, :] = jnp.zeros((HALF,), jnp.float32)
        return carry

    lax.fori_loop(0, CHUNK, zero_rows, 0)

    zbase = s * RPT
    for q in range(RPT // CHUNK):
        pltpu.sync_copy(rows_a, acc.at[pl.ds(zbase + q * CHUNK, CHUNK)])
    rem = RPT - (RPT // CHUNK) * CHUNK
    if rem:
        pltpu.sync_copy(rows_a.at[pl.ds(0, rem)],
                        acc.at[pl.ds(zbase + (RPT // CHUNK) * CHUNK, rem)])
    plsc.subcore_barrier()

    def fire_idx_and_gather(k, t):
        rb = s * (EPT // GG) + k * NG
        pltpu.async_copy(col2.at[pl.ds(c * (EE // GG) + rb, NG)], cols[t],
                         semis[t])
        pltpu.async_copy(val2.at[pl.ds(rb, NG)], vals[t], semis[t])
        pltpu.async_copy(row2.at[pl.ds(rb, NG)], rowfs[t], semis[t])
        pltpu.make_async_copy(col2.at[pl.ds(0, NG)], cols[t], semis[t]).wait()
        pltpu.make_async_copy(val2.at[pl.ds(0, NG)], vals[t], semis[t]).wait()
        pltpu.make_async_copy(row2.at[pl.ds(0, NG)], rowfs[t], semis[t]).wait()
        for j in range(NG):
            pltpu.async_copy(tbl.at[cols[t].at[j]],
                             rowss[t].at[pl.ds(j * GG, GG)], semgs[t])

    fire_idx_and_gather(0, 0)

    def outer(i, carry):
        for b in range(2):
            o = 1 - b
            k = 2 * i + b
            @pl.when(k >= 1)
            def _drain_scatter():
                pltpu.make_async_copy(tbl.at[pl.ds(0, CHUNK)], rowss[o],
                                      semss[o]).wait()

            @pl.when(k + 1 < NCHUNK)
            def _prefetch():
                fire_idx_and_gather(k + 1, o)

            pltpu.make_async_copy(tbl.at[pl.ds(0, CHUNK)], rowss[b],
                                  semgs[b]).wait()
            for j in range(NG):
                for t in range(GG // HALF):
                    vv = vals[b][j, pl.ds(t * HALF, HALF)]
                    for u in range(HALF):
                        r = j * GG + t * HALF + u
                        rowss[b][r, :] = rowss[b][r, :] * vv[u]
            for j in range(NG):
                pltpu.async_copy(rowss[b].at[pl.ds(j * GG, GG)],
                                 acc.at[rowfs[b].at[j]], semss[b], add=True)
        return carry

    lax.fori_loop(0, NCHUNK // 2, outer, 0)
    pltpu.make_async_copy(tbl.at[pl.ds(0, CHUNK)], rowss[(NCHUNK - 1) % 2],
                          semss[(NCHUNK - 1) % 2]).wait()
    plsc.subcore_barrier()
    obase = c * NP + s * RPT
    for q in range(RPT // CHUNK):
        pltpu.sync_copy(acc.at[pl.ds(zbase + q * CHUNK, CHUNK)],
                        out.at[pl.ds(obase + q * CHUNK, CHUNK)])
    if rem:
        pltpu.sync_copy(acc.at[pl.ds(zbase + (RPT // CHUNK) * CHUNK, rem)],
                        out.at[pl.ds(obase + (RPT // CHUNK) * CHUNK, rem)])


@functools.cache
def _make_spmv():
    return functools.partial(
        pl.kernel,
        out_type=jax.ShapeDtypeStruct((2 * NP, HALF), jnp.float32),
        mesh=plsc.VectorSubcoreMesh(core_axis_name="c", subcore_axis_name="s",
                                    num_cores=NC, num_subcores=NS),
        scratch_types=[
            pltpu.VMEM((NG, GG), jnp.int32),
            pltpu.VMEM((NG, GG), jnp.int32),
            pltpu.VMEM((NG, GG), jnp.float32),
            pltpu.VMEM((NG, GG), jnp.float32),
            pltpu.VMEM((NG, GG), jnp.int32),
            pltpu.VMEM((NG, GG), jnp.int32),
            pltpu.VMEM((CHUNK, HALF), jnp.float32),
            pltpu.VMEM((CHUNK, HALF), jnp.float32),
            pltpu.VMEM_SHARED((NP, HALF), jnp.float32),
            pltpu.SemaphoreType.DMA,
            pltpu.SemaphoreType.DMA,
            pltpu.SemaphoreType.DMA,
            pltpu.SemaphoreType.DMA,
            pltpu.SemaphoreType.DMA,
            pltpu.SemaphoreType.DMA,
        ],
        compiler_params=pltpu.CompilerParams(use_tc_tiling_on_sc=False),
    )(_spmv_body)


def _dense_body(sL, sR, eL, eR, w8, bias, seg, segT, ego_out, norm_out):
    sl = sL[0]
    sr = sR[0]
    el = eL[0]
    er = eR[0]
    bl = el * sl
    br = er * sr
    dot = functools.partial(jnp.dot, preferred_element_type=jnp.float32)
    xL = (dot(sl, w8[0]) + dot(sr, w8[1]) + dot(bl, w8[2]) + dot(br, w8[3])
          + bias[0, :])
    xR = (dot(sl, w8[4]) + dot(sr, w8[5]) + dot(bl, w8[6]) + dot(br, w8[7])
          + bias[1, :])
    aL = jnp.where(xL >= 0, xL, 0.2 * xL)
    aR = jnp.where(xR >= 0, xR, 0.2 * xR)
    hi = jax.lax.Precision.HIGHEST
    ss = (jnp.dot(aL * aL, seg[...], preferred_element_type=jnp.float32,
                  precision=hi)
          + jnp.dot(aR * aR, seg[...], preferred_element_type=jnp.float32,
                    precision=hi))
    inv = 1.0 / jnp.maximum(jnp.sqrt(ss), 1e-12)
    den = jnp.dot(inv, segT[...], preferred_element_type=jnp.float32,
                  precision=hi)
    ego_out[0] = aL
    ego_out[1] = aR
    norm_out[0] = aL * den
    norm_out[1] = aR * den


_dense = pl.pallas_call(
    _dense_body,
    grid=(RR // RB,),
    in_specs=[
        pl.BlockSpec((1, RB, 128), lambda i: (0, i, 0)),
        pl.BlockSpec((1, RB, 128), lambda i: (1, i, 0)),
        pl.BlockSpec((1, RB, 128), lambda i: (0, i, 0)),
        pl.BlockSpec((1, RB, 128), lambda i: (1, i, 0)),
        pl.BlockSpec((8, 128, 128), lambda i: (0, 0, 0)),
        pl.BlockSpec((2, 128), lambda i: (0, 0)),
        pl.BlockSpec((128, 8), lambda i: (0, 0)),
        pl.BlockSpec((8, 128), lambda i: (0, 0)),
    ],
    out_specs=[
        pl.BlockSpec((2, RB, 128), lambda i: (0, i, 0)),
        pl.BlockSpec((2, RB, 128), lambda i: (0, i, 0)),
    ],
    out_shape=[
        jax.ShapeDtypeStruct((2, RR, 128), jnp.float32),
        jax.ShapeDtypeStruct((2, RR, 128), jnp.float32),
    ],
)


def _gather_body(t0, t1, t2, t3, idx2, out, idxl, idxr, buf, sem):
    c = lax.axis_index("c")
    s = lax.axis_index("s")
    base = (s * NC + c) * GPW
    tabs = [t0, t1, t2, t3]
    for kset in range(3):
        pltpu.sync_copy(idx2.at[0, pl.ds(kset * BB + base, GPW)], idxl)
        pltpu.sync_copy(idx2.at[1, pl.ds(kset * BB + base, GPW)], idxr)
        for t in range(4):
            for h, iv in ((0, idxl), (1, idxr)):
                pltpu.async_copy(tabs[t].at[iv], buf, sem).wait()
                pltpu.sync_copy(
                    buf, out.at[kset, pl.ds(base, GPW),
                                pl.ds(t * DD + h * HALF, HALF)])


@functools.cache
def _make_batch_gather():
    return functools.partial(
        pl.kernel,
        out_type=jax.ShapeDtypeStruct((3, BB, 4 * DD), jnp.float32),
        mesh=plsc.VectorSubcoreMesh(core_axis_name="c", subcore_axis_name="s",
                                    num_cores=NC, num_subcores=NS),
        scratch_types=[
            pltpu.VMEM((GPW,), jnp.int32),
            pltpu.VMEM((GPW,), jnp.int32),
            pltpu.VMEM((GPW, HALF), jnp.float32),
            pltpu.SemaphoreType.DMA,
        ],
        compiler_params=pltpu.CompilerParams(use_tc_tiling_on_sc=False),
    )(_gather_body)


def kernel(user, item_i, item_j, user_emb, item_emb, adj_row, adj_col, adj_val,
           W_gc_0, b_gc_0, W_bi_0, b_bi_0,
           W_gc_1, b_gc_1, W_bi_1, b_bi_1,
           W_gc_2, b_gc_2, W_bi_2, b_bi_2):
    Ws = [(W_gc_0, b_gc_0, W_bi_0, b_bi_0),
          (W_gc_1, b_gc_1, W_bi_1, b_bi_1),
          (W_gc_2, b_gc_2, W_bi_2, b_bi_2)]
    ego0 = jnp.concatenate([user_emb, item_emb], axis=0)
    ego0p = jnp.pad(ego0, ((0, NP - NN), (0, 0)))
    ego_p = jnp.stack([ego0p[:, :HALF].reshape(RR, 128),
                       ego0p[:, HALF:].reshape(RR, 128)])
    col2 = jnp.concatenate([adj_col, adj_col + NP]).reshape(2 * EE // GG, GG)
    row2 = adj_row.reshape(EE // GG, GG)
    val2 = adj_val.reshape(EE // GG, GG)
    idx1 = jnp.concatenate([user.astype(jnp.int32),
                            item_i + NUSER,
                            item_j + NUSER])
    idx2 = jnp.stack([idx1, idx1 + NP])

    eye8 = jnp.eye(8, dtype=jnp.float32)
    lanes = jnp.arange(128, dtype=jnp.int32) // HALF
    seg = (lanes[:, None] == jnp.arange(8, dtype=jnp.int32)[None, :]
           ).astype(jnp.float32)
    segT = seg.T

    spmv = _make_spmv()
    ego0_tbl = ego_p.reshape(2 * NP, HALF)
    norms = []
    for (wg, bg, wb, bb) in Ws:
        w8 = jnp.stack([
            jnp.kron(eye8, wg[:HALF, :HALF]), jnp.kron(eye8, wg[HALF:, :HALF]),
            jnp.kron(eye8, wb[:HALF, :HALF]), jnp.kron(eye8, wb[HALF:, :HALF]),
            jnp.kron(eye8, wg[:HALF, HALF:]), jnp.kron(eye8, wg[HALF:, HALF:]),
            jnp.kron(eye8, wb[:HALF, HALF:]), jnp.kron(eye8, wb[HALF:, HALF:]),
        ])
        bsum = (bg + bb)[0]
        bias = jnp.stack([jnp.tile(bsum[:HALF], 8), jnp.tile(bsum[HALF:], 8)])
        side_pad = spmv(ego_p.reshape(2 * NP, HALF),
                        col2, row2, val2)
        side_p = side_pad.reshape(2, RR, 128)
        ego_p, norm_p = _dense(side_p, side_p, ego_p, ego_p,
                               w8, bias, seg, segT)
        norms.append(norm_p.reshape(2 * NP, HALF))

    out = _make_batch_gather()(ego0_tbl, norms[0], norms[1], norms[2], idx2)
    return (out[0], out[1], out[2])


# speedup vs baseline: 17.9137x; 1.0290x over previous
import functools

import jax
import jax.numpy as jnp
from jax import lax
from jax.experimental import pallas as pl
from jax.experimental.pallas import tpu as pltpu
from jax.experimental.pallas import tpu_sc as plsc

NUSER = 30000
NITEM = 70000
NN = NUSER + NITEM
NP = 100096
EE = 1600000
DD = 32
HALF = 16
BB = 4096

NC = 2
NS = 16
EPT = EE // NS
CHUNK = 400
NCHUNK = EPT // CHUNK
GG = 80
NG = CHUNK // GG
RPT = NP // NS
RR = NP // 8
RB = 3128
GPW = BB // (NC * NS)


def _spmv_body(tbl, col2, row2, val2, out,
               col_a, col_b, val_a, val_b, rowf_a, rowf_b, rows_a, rows_b,
               acc, semg_a, semg_b, semi_a, semi_b, sems_a, sems_b):
    c = lax.axis_index("c")
    s = lax.axis_index("s")
    cols = [col_a, col_b]
    vals = [val_a, val_b]
    rowfs = [rowf_a, rowf_b]
    rowss = [rows_a, rows_b]
    semgs = [semg_a, semg_b]
    semis = [semi_a, semi_b]
    semss = [sems_a, sems_b]

    def zero_rows(i, carry):
        rows_a[i, :] = jnp.zeros((HALF,), jnp.float32)
        return carry

    lax.fori_loop(0, CHUNK, zero_rows, 0)

    zbase = s * RPT
    for q in range(RPT // CHUNK):
        pltpu.sync_copy(rows_a, acc.at[pl.ds(zbase + q * CHUNK, CHUNK)])
    rem = RPT - (RPT // CHUNK) * CHUNK
    if rem:
        pltpu.sync_copy(rows_a.at[pl.ds(0, rem)],
                        acc.at[pl.ds(zbase + (RPT // CHUNK) * CHUNK, rem)])
    plsc.subcore_barrier()

    def fire_idx_and_gather(k, t):
        rb = s * (EPT // GG) + k * NG
        pltpu.async_copy(col2.at[pl.ds(rb, NG)], cols[t], semis[t])
        pltpu.async_copy(val2.at[pl.ds(rb, NG)], vals[t], semis[t])
        pltpu.async_copy(row2.at[pl.ds(rb, NG)], rowfs[t], semis[t])
        pltpu.make_async_copy(col2.at[pl.ds(0, NG)], cols[t], semis[t]).wait()
        pltpu.make_async_copy(val2.at[pl.ds(0, NG)], vals[t], semis[t]).wait()
        pltpu.make_async_copy(row2.at[pl.ds(0, NG)], rowfs[t], semis[t]).wait()
        off = c * NP
        for j in range(NG):
            cols[t][j, :] = cols[t][j, :] + off
        for j in range(NG):
            pltpu.async_copy(tbl.at[cols[t].at[j]],
                             rowss[t].at[pl.ds(j * GG, GG)], semgs[t])

    fire_idx_and_gather(0, 0)

    def outer(i, carry):
        for b in range(2):
            o = 1 - b
            k = 2 * i + b
            @pl.when(k >= 1)
            def _drain_scatter():
                pltpu.make_async_copy(tbl.at[pl.ds(0, CHUNK)], rowss[o],
                                      semss[o]).wait()

            @pl.when(k + 1 < NCHUNK)
            def _prefetch():
                fire_idx_and_gather(k + 1, o)

            pltpu.make_async_copy(tbl.at[pl.ds(0, CHUNK)], rowss[b],
                                  semgs[b]).wait()
            for j in range(NG):
                for t in range(GG // HALF):
                    vv = vals[b][j, pl.ds(t * HALF, HALF)]
                    for u in range(HALF):
                        r = j * GG + t * HALF + u
                        rowss[b][r, :] = rowss[b][r, :] * vv[u]
            for j in range(NG):
                pltpu.async_copy(rowss[b].at[pl.ds(j * GG, GG)],
                                 acc.at[rowfs[b].at[j]], semss[b], add=True)
        return carry

    lax.fori_loop(0, NCHUNK // 2, outer, 0)
    pltpu.make_async_copy(tbl.at[pl.ds(0, CHUNK)], rowss[(NCHUNK - 1) % 2],
                          semss[(NCHUNK - 1) % 2]).wait()
    plsc.subcore_barrier()
    obase = c * NP + s * RPT
    for q in range(RPT // CHUNK):
        pltpu.sync_copy(acc.at[pl.ds(zbase + q * CHUNK, CHUNK)],
                        out.at[pl.ds(obase + q * CHUNK, CHUNK)])
    if rem:
        pltpu.sync_copy(acc.at[pl.ds(zbase + (RPT // CHUNK) * CHUNK, rem)],
                        out.at[pl.ds(obase + (RPT // CHUNK) * CHUNK, rem)])


@functools.cache
def _make_spmv():
    return functools.partial(
        pl.kernel,
        out_type=jax.ShapeDtypeStruct((2 * NP, HALF), jnp.float32),
        mesh=plsc.VectorSubcoreMesh(core_axis_name="c", subcore_axis_name="s",
                                    num_cores=NC, num_subcores=NS),
        scratch_types=[
            pltpu.VMEM((NG, GG), jnp.int32),
            pltpu.VMEM((NG, GG), jnp.int32),
            pltpu.VMEM((NG, GG), jnp.float32),
            pltpu.VMEM((NG, GG), jnp.float32),
            pltpu.VMEM((NG, GG), jnp.int32),
            pltpu.VMEM((NG, GG), jnp.int32),
            pltpu.VMEM((CHUNK, HALF), jnp.float32),
            pltpu.VMEM((CHUNK, HALF), jnp.float32),
            pltpu.VMEM_SHARED((NP, HALF), jnp.float32),
            pltpu.SemaphoreType.DMA,
            pltpu.SemaphoreType.DMA,
            pltpu.SemaphoreType.DMA,
            pltpu.SemaphoreType.DMA,
            pltpu.SemaphoreType.DMA,
            pltpu.SemaphoreType.DMA,
        ],
        compiler_params=pltpu.CompilerParams(use_tc_tiling_on_sc=False),
    )(_spmv_body)


def _dense_body(sL, sR, eL, eR, w8, bias, seg, segT, ego_out, norm_out):
    sl = sL[0]
    sr = sR[0]
    el = eL[0]
    er = eR[0]
    bl = el * sl
    br = er * sr
    dot = functools.partial(jnp.dot, preferred_element_type=jnp.float32)
    xL = (dot(sl, w8[0]) + dot(sr, w8[1]) + dot(bl, w8[2]) + dot(br, w8[3])
          + bias[0, :])
    xR = (dot(sl, w8[4]) + dot(sr, w8[5]) + dot(bl, w8[6]) + dot(br, w8[7])
          + bias[1, :])
    aL = jnp.where(xL >= 0, xL, 0.2 * xL)
    aR = jnp.where(xR >= 0, xR, 0.2 * xR)
    hi = jax.lax.Precision.HIGHEST
    ss = (jnp.dot(aL * aL, seg[...], preferred_element_type=jnp.float32,
                  precision=hi)
          + jnp.dot(aR * aR, seg[...], preferred_element_type=jnp.float32,
                    precision=hi))
    inv = 1.0 / jnp.maximum(jnp.sqrt(ss), 1e-12)
    den = jnp.dot(inv, segT[...], preferred_element_type=jnp.float32,
                  precision=hi)
    ego_out[0] = aL
    ego_out[1] = aR
    norm_out[0] = aL * den
    norm_out[1] = aR * den


_dense = pl.pallas_call(
    _dense_body,
    grid=(RR // RB,),
    in_specs=[
        pl.BlockSpec((1, RB, 128), lambda i: (0, i, 0)),
        pl.BlockSpec((1, RB, 128), lambda i: (1, i, 0)),
        pl.BlockSpec((1, RB, 128), lambda i: (0, i, 0)),
        pl.BlockSpec((1, RB, 128), lambda i: (1, i, 0)),
        pl.BlockSpec((8, 128, 128), lambda i: (0, 0, 0)),
        pl.BlockSpec((2, 128), lambda i: (0, 0)),
        pl.BlockSpec((128, 8), lambda i: (0, 0)),
        pl.BlockSpec((8, 128), lambda i: (0, 0)),
    ],
    out_specs=[
        pl.BlockSpec((2, RB, 128), lambda i: (0, i, 0)),
        pl.BlockSpec((2, RB, 128), lambda i: (0, i, 0)),
    ],
    out_shape=[
        jax.ShapeDtypeStruct((2, RR, 128), jnp.float32),
        jax.ShapeDtypeStruct((2, RR, 128), jnp.float32),
    ],
)


def _gather_body(t0, t1, t2, t3, idx2, out, idxl, idxr, buf, sem):
    c = lax.axis_index("c")
    s = lax.axis_index("s")
    base = (s * NC + c) * GPW
    tabs = [t0, t1, t2, t3]
    for kset in range(3):
        pltpu.sync_copy(idx2.at[0, pl.ds(kset * BB + base, GPW)], idxl)
        pltpu.sync_copy(idx2.at[1, pl.ds(kset * BB + base, GPW)], idxr)
        for t in range(4):
            for h, iv in ((0, idxl), (1, idxr)):
                pltpu.async_copy(tabs[t].at[iv], buf, sem).wait()
                pltpu.sync_copy(
                    buf, out.at[kset, pl.ds(base, GPW),
                                pl.ds(t * DD + h * HALF, HALF)])


@functools.cache
def _make_batch_gather():
    return functools.partial(
        pl.kernel,
        out_type=jax.ShapeDtypeStruct((3, BB, 4 * DD), jnp.float32),
        mesh=plsc.VectorSubcoreMesh(core_axis_name="c", subcore_axis_name="s",
                                    num_cores=NC, num_subcores=NS),
        scratch_types=[
            pltpu.VMEM((GPW,), jnp.int32),
            pltpu.VMEM((GPW,), jnp.int32),
            pltpu.VMEM((GPW, HALF), jnp.float32),
            pltpu.SemaphoreType.DMA,
        ],
        compiler_params=pltpu.CompilerParams(use_tc_tiling_on_sc=False),
    )(_gather_body)


def kernel(user, item_i, item_j, user_emb, item_emb, adj_row, adj_col, adj_val,
           W_gc_0, b_gc_0, W_bi_0, b_bi_0,
           W_gc_1, b_gc_1, W_bi_1, b_bi_1,
           W_gc_2, b_gc_2, W_bi_2, b_bi_2):
    Ws = [(W_gc_0, b_gc_0, W_bi_0, b_bi_0),
          (W_gc_1, b_gc_1, W_bi_1, b_bi_1),
          (W_gc_2, b_gc_2, W_bi_2, b_bi_2)]
    ego0 = jnp.concatenate([user_emb, item_emb], axis=0)
    ego0p = jnp.pad(ego0, ((0, NP - NN), (0, 0)))
    ego_p = jnp.stack([ego0p[:, :HALF].reshape(RR, 128),
                       ego0p[:, HALF:].reshape(RR, 128)])
    col2 = adj_col.reshape(EE // GG, GG)
    row2 = adj_row.reshape(EE // GG, GG)
    val2 = adj_val.reshape(EE // GG, GG)
    idx1 = jnp.concatenate([user.astype(jnp.int32),
                            item_i + NUSER,
                            item_j + NUSER])
    idx2 = jnp.stack([idx1, idx1 + NP])

    eye8 = jnp.eye(8, dtype=jnp.float32)
    lanes = jnp.arange(128, dtype=jnp.int32) // HALF
    seg = (lanes[:, None] == jnp.arange(8, dtype=jnp.int32)[None, :]
           ).astype(jnp.float32)
    segT = seg.T

    spmv = _make_spmv()
    ego0_tbl = ego_p.reshape(2 * NP, HALF)
    norms = []
    for (wg, bg, wb, bb) in Ws:
        w8 = jnp.stack([
            jnp.kron(eye8, wg[:HALF, :HALF]), jnp.kron(eye8, wg[HALF:, :HALF]),
            jnp.kron(eye8, wb[:HALF, :HALF]), jnp.kron(eye8, wb[HALF:, :HALF]),
            jnp.kron(eye8, wg[:HALF, HALF:]), jnp.kron(eye8, wg[HALF:, HALF:]),
            jnp.kron(eye8, wb[:HALF, HALF:]), jnp.kron(eye8, wb[HALF:, HALF:]),
        ])
        bsum = (bg + bb)[0]
        bias = jnp.stack([jnp.tile(bsum[:HALF], 8), jnp.tile(bsum[HALF:], 8)])
        side_pad = spmv(ego_p.reshape(2 * NP, HALF),
                        col2, row2, val2)
        side_p = side_pad.reshape(2, RR, 128)
        ego_p, norm_p = _dense(side_p, side_p, ego_p, ego_p,
                               w8, bias, seg, segT)
        norms.append(norm_p.reshape(2 * NP, HALF))

    out = _make_batch_gather()(ego0_tbl, norms[0], norms[1], norms[2], idx2)
    return (out[0], out[1], out[2])
